# Initial kernel scaffold; baseline (speedup 1.0000x reference)
import jax
import jax.numpy as jnp
from jax.experimental import pallas as pl


def kernel(x, keys_a, keys_b, values, W_q, W_g, b_g):
    raise NotImplementedError("write your pallas kernel here")


# speedup vs baseline: 1.4093x; 1.4093x over previous
import functools

import jax
import jax.numpy as jnp
from jax import lax
from jax.experimental import pallas as pl
from jax.experimental.pallas import tpu as pltpu
from jax.experimental.pallas import tpu_sc as plsc

N_SUBKEYS = 256
D_KEY = 64
TOP_K = 8

TB = 256
NEG_INF = float("-inf")


def _top8(s, iota):
    vals, idxs = [], []
    big = jnp.int32(1 << 30)
    for _ in range(TOP_K):
        m = jnp.max(s, axis=1, keepdims=True)
        eq = s == m
        i = jnp.min(jnp.where(eq, iota, big), axis=1, keepdims=True)
        vals.append(m)
        idxs.append(i)
        s = jnp.where(iota == i, NEG_INF, s)
    return vals, idxs


def _score_kernel(x_ref, wq_ref, ka_ref, kb_ref, wg_ref, bg_ref,
                  idx_ref, wts_ref):
    x = x_ref[...]
    q = lax.dot_general(x, wq_ref[...], (((1,), (1,)), ((), ())),
                        preferred_element_type=jnp.float32)
    qa = q[:, :D_KEY]
    qb = q[:, D_KEY:]
    sa = lax.dot_general(qa, ka_ref[...], (((1,), (1,)), ((), ())),
                         preferred_element_type=jnp.float32)
    sb = lax.dot_general(qb, kb_ref[...], (((1,), (1,)), ((), ())),
                         preferred_element_type=jnp.float32)

    iota256 = lax.broadcasted_iota(jnp.int32, (TB, N_SUBKEYS), 1)
    va, ia = _top8(sa, iota256)
    vb, ib = _top8(sb, iota256)
    vb_cat = jnp.concatenate(vb, axis=1)
    ib_cat = jnp.concatenate(ib, axis=1)

    comb = jnp.concatenate([va[p] + vb_cat for p in range(TOP_K)], axis=1)
    flat = jnp.concatenate([ia[p] * N_SUBKEYS + ib_cat for p in range(TOP_K)],
                           axis=1)

    iota64 = lax.broadcasted_iota(jnp.int32, (TB, TOP_K * TOP_K), 1)
    big = jnp.int32(1 << 30)
    fvals, fidxs = [], []
    for _ in range(TOP_K):
        m = jnp.max(comb, axis=1, keepdims=True)
        eq = comb == m
        pos = jnp.min(jnp.where(eq, iota64, big), axis=1, keepdims=True)
        sel = iota64 == pos
        fvals.append(m)
        fidxs.append(jnp.max(jnp.where(sel, flat, -1), axis=1, keepdims=True))
        comb = jnp.where(sel, NEG_INF, comb)

    exps = [jnp.exp(v - fvals[0]) for v in fvals]
    denom = exps[0]
    for e in exps[1:]:
        denom = denom + e

    glin = jnp.sum(x * wg_ref[...], axis=1, keepdims=True) + bg_ref[0, 0]
    g = jax.nn.sigmoid(glin)
    scale = g / denom

    idx_ref[...] = jnp.concatenate(fidxs, axis=1)
    wts_ref[...] = jnp.concatenate(
        [jnp.broadcast_to(e * scale, (TB, 16)) for e in exps], axis=1)


def _run_score(x2, wq, ka, kb, wg, bg):
    n_tok = x2.shape[0]
    d = x2.shape[1]
    grid = n_tok // TB
    return pl.pallas_call(
        _score_kernel,
        grid=(grid,),
        in_specs=[
            pl.BlockSpec((TB, d), lambda i: (i, 0)),
            pl.BlockSpec((2 * D_KEY, d), lambda i: (0, 0)),
            pl.BlockSpec((N_SUBKEYS, D_KEY), lambda i: (0, 0)),
            pl.BlockSpec((N_SUBKEYS, D_KEY), lambda i: (0, 0)),
            pl.BlockSpec((1, d), lambda i: (0, 0)),
            pl.BlockSpec((1, 1), lambda i: (0, 0)),
        ],
        out_specs=[
            pl.BlockSpec((TB, TOP_K), lambda i: (i, 0)),
            pl.BlockSpec((TB, 16 * TOP_K), lambda i: (i, 0)),
        ],
        out_shape=[
            jax.ShapeDtypeStruct((n_tok, TOP_K), jnp.int32),
            jax.ShapeDtypeStruct((n_tok, 16 * TOP_K), jnp.float32),
        ],
    )(x2, wq, ka, kb, wg, bg)


C = 4
ROWS_C = C * TOP_K


def _sc_body(n_tok, d, values_hbm, idx_hbm, w_hbm, x_hbm, out_hbm,
             idx_v, w_v, rows_v, x_v, out_v, gsem, xsem, osem):
    info = plsc.get_sparse_core_info()
    nc = info.num_cores
    tpt = n_tok // (nc * info.num_subcores)
    wid = lax.axis_index("s") * nc + lax.axis_index("c")
    base_tok = wid * tpt

    pltpu.sync_copy(w_hbm.at[pl.ds(base_tok, tpt)], w_v)

    n_chunks = tpt // C
    nd = d // 16

    def chunk_body(j, _):
        tok0 = base_tok + j * C
        pltpu.sync_copy(idx_hbm.at[pl.ds(tok0 * TOP_K, ROWS_C)], idx_v)
        cp_g = pltpu.async_copy(values_hbm.at[idx_v], rows_v, gsem)
        cp_x = pltpu.async_copy(x_hbm.at[pl.ds(tok0, C)], x_v, xsem)
        cp_x.wait()
        cp_g.wait()
        for t in range(C):
            wvecs = [w_v[j * C + t, pl.ds(16 * k, 16)] for k in range(TOP_K)]

            def dblk(b, _):
                off = pl.ds(b * 16, 16)
                acc = x_v[t, off]
                for k in range(TOP_K):
                    acc = acc + wvecs[k] * rows_v[t * TOP_K + k, off]
                out_v[t, off] = acc
                return 0

            lax.fori_loop(0, nd, dblk, 0, unroll=4)
        pltpu.sync_copy(out_v, out_hbm.at[pl.ds(tok0, C)])
        return 0

    lax.fori_loop(0, n_chunks, chunk_body, 0)


def _run_sc(values, idx_flat, w_flat, x2):
    n_tok, d = x2.shape
    mesh = plsc.VectorSubcoreMesh(core_axis_name="c", subcore_axis_name="s")
    body = functools.partial(_sc_body, n_tok, d)
    kern = pl.kernel(
        body,
        out_type=jax.ShapeDtypeStruct((n_tok, d), jnp.float32),
        mesh=mesh,
        scratch_types=[
            pltpu.VMEM((ROWS_C,), jnp.int32),
            pltpu.VMEM((n_tok // 32, 16 * TOP_K), jnp.float32),
            pltpu.VMEM((ROWS_C, d), jnp.float32),
            pltpu.VMEM((C, d), jnp.float32),
            pltpu.VMEM((C, d), jnp.float32),
            pltpu.SemaphoreType.DMA,
            pltpu.SemaphoreType.DMA,
            pltpu.SemaphoreType.DMA,
        ],
    )
    return kern(values, idx_flat, w_flat, x2)


def kernel(x, keys_a, keys_b, values, W_q, W_g, b_g):
    B, T, D = x.shape
    x2 = x.reshape(B * T, D)
    idx, wts = _run_score(x2, W_q, keys_a, keys_b, W_g,
                          b_g.reshape(1, 1))
    out = _run_sc(values, idx.reshape(-1), wts, x2)
    return out.reshape(B, T, D)


# speedup vs baseline: 1.8496x; 1.3125x over previous
import functools

import jax
import jax.numpy as jnp
from jax import lax
from jax.experimental import pallas as pl
from jax.experimental.pallas import tpu as pltpu
from jax.experimental.pallas import tpu_sc as plsc

N_SUBKEYS = 256
D_KEY = 64
TOP_K = 8

TB = 256
NEG_INF = float("-inf")


def _top8(s, iota):
    vals, idxs = [], []
    big = jnp.int32(1 << 30)
    for _ in range(TOP_K):
        m = jnp.max(s, axis=1, keepdims=True)
        eq = s == m
        i = jnp.min(jnp.where(eq, iota, big), axis=1, keepdims=True)
        vals.append(m)
        idxs.append(i)
        s = jnp.where(iota == i, NEG_INF, s)
    return vals, idxs


def _score_kernel(x_ref, wq_ref, ka_ref, kb_ref, wg_ref, bg_ref,
                  idx_ref, wts_ref):
    x = x_ref[...]
    q = lax.dot_general(x, wq_ref[...], (((1,), (1,)), ((), ())),
                        preferred_element_type=jnp.float32)
    qa = q[:, :D_KEY]
    qb = q[:, D_KEY:]
    sa = lax.dot_general(qa, ka_ref[...], (((1,), (1,)), ((), ())),
                         preferred_element_type=jnp.float32)
    sb = lax.dot_general(qb, kb_ref[...], (((1,), (1,)), ((), ())),
                         preferred_element_type=jnp.float32)

    iota256 = lax.broadcasted_iota(jnp.int32, (TB, N_SUBKEYS), 1)
    va, ia = _top8(sa, iota256)
    vb, ib = _top8(sb, iota256)
    vb_cat = jnp.concatenate(vb, axis=1)
    ib_cat = jnp.concatenate(ib, axis=1)

    comb = jnp.concatenate([va[p] + vb_cat for p in range(TOP_K)], axis=1)
    flat = jnp.concatenate([ia[p] * N_SUBKEYS + ib_cat for p in range(TOP_K)],
                           axis=1)

    iota64 = lax.broadcasted_iota(jnp.int32, (TB, TOP_K * TOP_K), 1)
    big = jnp.int32(1 << 30)
    fvals, fidxs = [], []
    for _ in range(TOP_K):
        m = jnp.max(comb, axis=1, keepdims=True)
        eq = comb == m
        pos = jnp.min(jnp.where(eq, iota64, big), axis=1, keepdims=True)
        sel = iota64 == pos
        fvals.append(m)
        fidxs.append(jnp.max(jnp.where(sel, flat, -1), axis=1, keepdims=True))
        comb = jnp.where(sel, NEG_INF, comb)

    exps = [jnp.exp(v - fvals[0]) for v in fvals]
    denom = exps[0]
    for e in exps[1:]:
        denom = denom + e

    glin = jnp.sum(x * wg_ref[...], axis=1, keepdims=True) + bg_ref[0, 0]
    g = jax.nn.sigmoid(glin)
    scale = g / denom

    idx_ref[...] = jnp.concatenate(fidxs, axis=1)
    wts_ref[...] = jnp.concatenate(
        [jnp.broadcast_to(e * scale, (TB, 16)) for e in exps], axis=1)


def _run_score(x2, wq, ka, kb, wg, bg):
    n_tok = x2.shape[0]
    d = x2.shape[1]
    grid = n_tok // TB
    return pl.pallas_call(
        _score_kernel,
        grid=(grid,),
        in_specs=[
            pl.BlockSpec((TB, d), lambda i: (i, 0)),
            pl.BlockSpec((2 * D_KEY, d), lambda i: (0, 0)),
            pl.BlockSpec((N_SUBKEYS, D_KEY), lambda i: (0, 0)),
            pl.BlockSpec((N_SUBKEYS, D_KEY), lambda i: (0, 0)),
            pl.BlockSpec((1, d), lambda i: (0, 0)),
            pl.BlockSpec((1, 1), lambda i: (0, 0)),
        ],
        out_specs=[
            pl.BlockSpec((TB, TOP_K), lambda i: (i, 0)),
            pl.BlockSpec((TB, 16 * TOP_K), lambda i: (i, 0)),
        ],
        out_shape=[
            jax.ShapeDtypeStruct((n_tok, TOP_K), jnp.int32),
            jax.ShapeDtypeStruct((n_tok, 16 * TOP_K), jnp.float32),
        ],
    )(x2, wq, ka, kb, wg, bg)


C = 4
ROWS_C = C * TOP_K


def _sc_body(n_tok, d, values_hbm, idx_hbm, w_hbm, x_hbm, out_hbm,
             idx_v, w_v, rows_v, x_v, out_v,
             gsem0, gsem1, xsem0, xsem1, osem0, osem1):
    info = plsc.get_sparse_core_info()
    nc = info.num_cores
    tpt = n_tok // (nc * info.num_subcores)
    wid = lax.axis_index("s") * nc + lax.axis_index("c")
    base_tok = wid * tpt
    gsem = (gsem0, gsem1)
    xsem = (xsem0, xsem1)
    osem = (osem0, osem1)

    pltpu.sync_copy(idx_hbm.at[pl.ds(base_tok * TOP_K, tpt * TOP_K)], idx_v)
    pltpu.sync_copy(w_hbm.at[pl.ds(base_tok, tpt)], w_v)

    n_chunks = tpt // C
    nd = d // 16

    def issue(c, slot):
        tok0 = base_tok + c * C
        pltpu.async_copy(values_hbm.at[idx_v.at[pl.ds(c * ROWS_C, ROWS_C)]],
                         rows_v.at[slot], gsem[slot])
        pltpu.async_copy(x_hbm.at[pl.ds(tok0, C)], x_v.at[slot], xsem[slot])

    def wait_in(slot):
        pltpu.make_async_copy(values_hbm.at[pl.ds(0, ROWS_C)],
                              rows_v.at[slot], gsem[slot]).wait()
        pltpu.make_async_copy(x_hbm.at[pl.ds(0, C)], x_v.at[slot],
                              xsem[slot]).wait()

    def wait_out(slot):
        pltpu.make_async_copy(out_v.at[slot], out_hbm.at[pl.ds(0, C)],
                              osem[slot]).wait()

    def compute_store(c, slot):
        tok0 = base_tok + c * C
        for t in range(C):
            wvecs = [w_v[c * C + t, pl.ds(16 * k, 16)] for k in range(TOP_K)]

            def dblk(b, _):
                off = pl.ds(b * 16, 16)
                acc = x_v[slot, t, off]
                for k in range(TOP_K):
                    acc = acc + wvecs[k] * rows_v[slot, t * TOP_K + k, off]
                out_v[slot, t, off] = acc
                return 0

            lax.fori_loop(0, nd, dblk, 0, unroll=4)
        pltpu.async_copy(out_v.at[slot], out_hbm.at[pl.ds(tok0, C)],
                         osem[slot])

    issue(0, 0)
    issue(1, 1)
    for slot in (0, 1):
        wait_in(slot)
        compute_store(slot, slot)
        issue(slot + 2, slot)

    def steady(m, _):
        for slot in (0, 1):
            c = 2 * m + slot
            wait_in(slot)
            wait_out(slot)
            compute_store(c, slot)
            issue(c + 2, slot)
        return 0

    lax.fori_loop(1, n_chunks // 2 - 1, steady, 0)

    for slot in (0, 1):
        c = n_chunks - 2 + slot
        wait_in(slot)
        wait_out(slot)
        compute_store(c, slot)
    for slot in (0, 1):
        wait_out(slot)


def _run_sc(values, idx_flat, w_flat, x2):
    n_tok, d = x2.shape
    mesh = plsc.VectorSubcoreMesh(core_axis_name="c", subcore_axis_name="s")
    body = functools.partial(_sc_body, n_tok, d)
    kern = pl.kernel(
        body,
        out_type=jax.ShapeDtypeStruct((n_tok, d), jnp.float32),
        mesh=mesh,
        scratch_types=[
            pltpu.VMEM((n_tok // 32 * TOP_K,), jnp.int32),
            pltpu.VMEM((n_tok // 32, 16 * TOP_K), jnp.float32),
            pltpu.VMEM((2, ROWS_C, d), jnp.float32),
            pltpu.VMEM((2, C, d), jnp.float32),
            pltpu.VMEM((2, C, d), jnp.float32),
            pltpu.SemaphoreType.DMA,
            pltpu.SemaphoreType.DMA,
            pltpu.SemaphoreType.DMA,
            pltpu.SemaphoreType.DMA,
            pltpu.SemaphoreType.DMA,
            pltpu.SemaphoreType.DMA,
        ],
    )
    return kern(values, idx_flat, w_flat, x2)


def kernel(x, keys_a, keys_b, values, W_q, W_g, b_g):
    B, T, D = x.shape
    x2 = x.reshape(B * T, D)
    idx, wts = _run_score(x2, W_q, keys_a, keys_b, W_g,
                          b_g.reshape(1, 1))
    out = _run_sc(values, idx.reshape(-1), wts, x2)
    return out.reshape(B, T, D)
